# baseline (speedup 1.0000x reference)
import jax
import jax.numpy as jnp
import numpy as np
from jax import lax
from jax.experimental import pallas as pl
from jax.experimental.pallas import tpu as pltpu

_TINY = np.float32(1.1754943508222875e-38)
_K0 = 0
_K1 = 42
_K2 = _K0 ^ _K1 ^ 0x1BD11BDA


def _rotl(x, d):
    return lax.shift_left(x, jnp.int32(d)) | lax.shift_right_logical(
        x, jnp.int32(32 - d))


def _threefry_bits(i):
    ks = [jnp.int32(_K0), jnp.int32(_K1), jnp.int32(_K2)]
    rot0 = (13, 15, 26, 6)
    rot1 = (17, 29, 16, 24)
    x0 = jnp.zeros_like(i) + ks[0]
    x1 = i + ks[1]

    def rounds(x0, x1, rots):
        for r in rots:
            x0 = x0 + x1
            x1 = x0 ^ _rotl(x1, r)
        return x0, x1

    x0, x1 = rounds(x0, x1, rot0)
    x0 = x0 + ks[1]
    x1 = x1 + (ks[2] + jnp.int32(1))
    x0, x1 = rounds(x0, x1, rot1)
    x0 = x0 + ks[2]
    x1 = x1 + (ks[0] + jnp.int32(2))
    x0, x1 = rounds(x0, x1, rot0)
    x0 = x0 + ks[0]
    x1 = x1 + (ks[1] + jnp.int32(3))
    x0, x1 = rounds(x0, x1, rot1)
    x0 = x0 + ks[1]
    x1 = x1 + (ks[2] + jnp.int32(4))
    x0, x1 = rounds(x0, x1, rot0)
    x0 = x0 + ks[2]
    x1 = x1 + (ks[0] + jnp.int32(5))
    return x0 ^ x1


def _gumbel_from_bits(bits):
    fbits = lax.shift_right_logical(bits, jnp.int32(9)) | jnp.int32(0x3F800000)
    f = lax.bitcast_convert_type(fbits, jnp.float32) - np.float32(1.0)
    span = np.float32(np.float32(1.0) - _TINY)
    u = jnp.maximum(_TINY, f * span + _TINY)
    return -jnp.log(-jnp.log(u))


def _stats_kernel(x_ref, mu_ref, den_ref):
    s = jnp.zeros((128, 128), jnp.float32)
    q = jnp.zeros((128, 128), jnp.float32)
    for c in range(128):
        v = x_ref[:, c * 128:(c + 1) * 128]
        s = s + v
        q = q + v * v
    for sh in (8, 16, 32, 64):
        s = s + pltpu.roll(s, sh, axis=1)
        q = q + pltpu.roll(q, sh, axis=1)
    s3 = s.reshape(8, 16, 128)
    q3 = q.reshape(8, 16, 128)
    grp = lax.broadcasted_iota(jnp.int32, (8, 16, 128), 2) >> 3
    mid = lax.broadcasted_iota(jnp.int32, (8, 16, 128), 1)
    sel = mid == grp
    s_t = jnp.sum(jnp.where(sel, s3, 0.0), axis=1)
    q_t = jnp.sum(jnp.where(sel, q3, 0.0), axis=1)
    inv_n = jnp.float32(1.0 / 2048.0)
    mu = s_t * inv_n
    var = q_t * inv_n - mu * mu
    den = jnp.sqrt(jnp.maximum(var, 0.0)) + jnp.float32(0.001)
    mu_ref[...] = mu
    den_ref[...] = den


def _sample_kernel(y_ref, mu_ref, den_ref, out_ref):
    r0 = pl.program_id(0) * 1024
    m = jnp.full((128, 8, 128), -jnp.inf, jnp.float32)
    jwin = jnp.zeros((128, 8, 128), jnp.int32)
    for c in range(16):
        v3 = y_ref[:, c * 128:(c + 1) * 128].reshape(128, 8, 128)
        mu_t = mu_ref[c].reshape(1, 8, 128)
        den_t = den_ref[c].reshape(1, 8, 128)
        std = (v3 - mu_t) / den_t
        a_i = lax.broadcasted_iota(jnp.int32, (128, 8, 128), 0)
        s_i = lax.broadcasted_iota(jnp.int32, (128, 8, 128), 1)
        l_i = lax.broadcasted_iota(jnp.int32, (128, 8, 128), 2)
        lin = (r0 + a_i * 8 + s_i) * 2048 + (c * 128) + l_i
        g = _gumbel_from_bits(_threefry_bits(lin))
        val = std + g
        upd = val > m
        m = jnp.where(upd, val, m)
        jwin = jnp.where(upd, jnp.int32(c * 128) + l_i, jwin)
    row_max = jnp.max(m, axis=2, keepdims=True)
    cand = jnp.where(m == row_max, jwin, jnp.int32(0x7FFFFFFF))
    out_ref[...] = jnp.min(cand, axis=2)


@jax.jit
def kernel(x):
    B, H, W = x.shape
    y2 = x.reshape(B, H * W)
    mu128, den128 = pl.pallas_call(
        _stats_kernel,
        grid=(16,),
        in_specs=[pl.BlockSpec((128, H * W), lambda k: (k, 0))],
        out_specs=[
            pl.BlockSpec((8, 128), lambda k: (k, 0)),
            pl.BlockSpec((8, 128), lambda k: (k, 0)),
        ],
        out_shape=[
            jax.ShapeDtypeStruct((128, 128), jnp.float32),
            jax.ShapeDtypeStruct((128, 128), jnp.float32),
        ],
    )(y2)
    mu3 = mu128.reshape(8, 16, 128).transpose(1, 0, 2)
    den3 = den128.reshape(8, 16, 128).transpose(1, 0, 2)

    y = x.reshape(B * W, H)
    out = pl.pallas_call(
        _sample_kernel,
        grid=(16,),
        in_specs=[
            pl.BlockSpec((1024, H), lambda k: (k, 0)),
            pl.BlockSpec((16, 8, 128), lambda k: (0, 0, 0)),
            pl.BlockSpec((16, 8, 128), lambda k: (0, 0, 0)),
        ],
        out_specs=pl.BlockSpec((128, 8), lambda k: (k, 0)),
        out_shape=jax.ShapeDtypeStruct((B, W), jnp.int32),
    )(y, mu3, den3)
    return out.reshape(1, H, W)


# speedup vs baseline: 2.6298x; 2.6298x over previous
import jax
import jax.numpy as jnp
import numpy as np
from jax import lax
from jax.experimental import pallas as pl
from jax.experimental.pallas import tpu as pltpu

_TINY = np.float32(1.1754943508222875e-38)
_K0 = 0
_K1 = 42
_K2 = _K0 ^ _K1 ^ 0x1BD11BDA


def _rotl(x, d):
    return lax.shift_left(x, jnp.int32(d)) | lax.shift_right_logical(
        x, jnp.int32(32 - d))


def _threefry_bits(i):
    ks = [jnp.int32(_K0), jnp.int32(_K1), jnp.int32(_K2)]
    rot0 = (13, 15, 26, 6)
    rot1 = (17, 29, 16, 24)
    x0 = jnp.zeros_like(i) + ks[0]
    x1 = i + ks[1]

    def rounds(x0, x1, rots):
        for r in rots:
            x0 = x0 + x1
            x1 = x0 ^ _rotl(x1, r)
        return x0, x1

    x0, x1 = rounds(x0, x1, rot0)
    x0 = x0 + ks[1]
    x1 = x1 + (ks[2] + jnp.int32(1))
    x0, x1 = rounds(x0, x1, rot1)
    x0 = x0 + ks[2]
    x1 = x1 + (ks[0] + jnp.int32(2))
    x0, x1 = rounds(x0, x1, rot0)
    x0 = x0 + ks[0]
    x1 = x1 + (ks[1] + jnp.int32(3))
    x0, x1 = rounds(x0, x1, rot1)
    x0 = x0 + ks[1]
    x1 = x1 + (ks[2] + jnp.int32(4))
    x0, x1 = rounds(x0, x1, rot0)
    x0 = x0 + ks[2]
    x1 = x1 + (ks[0] + jnp.int32(5))
    return x0 ^ x1


def _gumbel_from_bits(bits):
    fbits = lax.shift_right_logical(bits, jnp.int32(9)) | jnp.int32(0x3F800000)
    f = lax.bitcast_convert_type(fbits, jnp.float32) - np.float32(1.0)
    span = np.float32(np.float32(1.0) - _TINY)
    u = jnp.maximum(_TINY, f * span + _TINY)
    return -jnp.log(-jnp.log(u))


def _stats_kernel(x_ref, mu_ref, inv_ref):
    s = jnp.zeros((128, 128), jnp.float32)
    q = jnp.zeros((128, 128), jnp.float32)
    for c in range(128):
        v = x_ref[:, c * 128:(c + 1) * 128]
        s = s + v
        q = q + v * v
    for sh in (8, 16, 32, 64):
        s = s + pltpu.roll(s, sh, axis=1)
        q = q + pltpu.roll(q, sh, axis=1)
    s3 = s.reshape(8, 16, 128)
    q3 = q.reshape(8, 16, 128)
    grp = lax.broadcasted_iota(jnp.int32, (8, 16, 128), 2) >> 3
    mid = lax.broadcasted_iota(jnp.int32, (8, 16, 128), 1)
    sel = mid == grp
    s_t = jnp.sum(jnp.where(sel, s3, 0.0), axis=1)
    q_t = jnp.sum(jnp.where(sel, q3, 0.0), axis=1)
    inv_n = jnp.float32(1.0 / 2048.0)
    mu = s_t * inv_n
    var = q_t * inv_n - mu * mu
    den = jnp.sqrt(jnp.maximum(var, 0.0)) + jnp.float32(0.001)
    mu_ref[...] = mu
    inv_ref[...] = jnp.float32(1.0) / den


def _sample_kernel(y_ref, mu_ref, inv_ref, out_ref):
    a0 = pl.program_id(0) * 128
    a_i = lax.broadcasted_iota(jnp.int32, (128, 128), 0)
    l_i = lax.broadcasted_iota(jnp.int32, (128, 128), 1)
    base = (a0 + a_i) * jnp.int32(16384) + l_i
    cols = []
    for t in range(8):
        m = jnp.full((128, 128), -jnp.inf, jnp.float32)
        jw = jnp.zeros((128, 128), jnp.int32)
        for c in range(16):
            off = t * 2048 + c * 128
            v = y_ref[:, off:off + 128]
            mu_r = mu_ref[t * 16 + c, :].reshape(1, 128)
            inv_r = inv_ref[t * 16 + c, :].reshape(1, 128)
            std = (v - mu_r) * inv_r
            g = _gumbel_from_bits(_threefry_bits(base + jnp.int32(off)))
            val = std + g
            upd = val > m
            m = jnp.where(upd, val, m)
            jw = jnp.where(upd, jnp.int32(c * 128) + l_i, jw)
        row_max = jnp.max(m, axis=1, keepdims=True)
        cand = jnp.where(m == row_max, jw, jnp.int32(0x7FFFFFFF))
        cols.append(jnp.min(cand, axis=1, keepdims=True))
    out_ref[...] = jnp.concatenate(cols, axis=1)


@jax.jit
def kernel(x):
    B, H, W = x.shape
    y2 = x.reshape(B, H * W)
    mu128, inv128 = pl.pallas_call(
        _stats_kernel,
        grid=(16,),
        in_specs=[pl.BlockSpec((128, H * W), lambda k: (k, 0))],
        out_specs=[
            pl.BlockSpec((8, 128), lambda k: (k, 0)),
            pl.BlockSpec((8, 128), lambda k: (k, 0)),
        ],
        out_shape=[
            jax.ShapeDtypeStruct((128, 128), jnp.float32),
            jax.ShapeDtypeStruct((128, 128), jnp.float32),
        ],
    )(y2)
    out = pl.pallas_call(
        _sample_kernel,
        grid=(16,),
        in_specs=[
            pl.BlockSpec((128, H * W), lambda k: (k, 0)),
            pl.BlockSpec((128, 128), lambda k: (0, 0)),
            pl.BlockSpec((128, 128), lambda k: (0, 0)),
        ],
        out_specs=pl.BlockSpec((128, 8), lambda k: (k, 0)),
        out_shape=jax.ShapeDtypeStruct((B, W), jnp.int32),
    )(y2, mu128, inv128)
    return out.reshape(1, H, W)


# speedup vs baseline: 3.1803x; 1.2094x over previous
import jax
import jax.numpy as jnp
import numpy as np
from jax import lax
from jax.experimental import pallas as pl

_TINY = np.float32(1.1754943508222875e-38)
_K0 = 0
_K1 = 42
_K2 = _K0 ^ _K1 ^ 0x1BD11BDA


def _rotl(x, d):
    return lax.shift_left(x, jnp.int32(d)) | lax.shift_right_logical(
        x, jnp.int32(32 - d))


def _threefry_bits(i):
    ks = [jnp.int32(_K0), jnp.int32(_K1), jnp.int32(_K2)]
    rot0 = (13, 15, 26, 6)
    rot1 = (17, 29, 16, 24)
    x0 = jnp.zeros_like(i) + ks[0]
    x1 = i + ks[1]

    def rounds(x0, x1, rots):
        for r in rots:
            x0 = x0 + x1
            x1 = x0 ^ _rotl(x1, r)
        return x0, x1

    x0, x1 = rounds(x0, x1, rot0)
    x0 = x0 + ks[1]
    x1 = x1 + (ks[2] + jnp.int32(1))
    x0, x1 = rounds(x0, x1, rot1)
    x0 = x0 + ks[2]
    x1 = x1 + (ks[0] + jnp.int32(2))
    x0, x1 = rounds(x0, x1, rot0)
    x0 = x0 + ks[0]
    x1 = x1 + (ks[1] + jnp.int32(3))
    x0, x1 = rounds(x0, x1, rot1)
    x0 = x0 + ks[1]
    x1 = x1 + (ks[2] + jnp.int32(4))
    x0, x1 = rounds(x0, x1, rot0)
    x0 = x0 + ks[2]
    x1 = x1 + (ks[0] + jnp.int32(5))
    return x0 ^ x1


def _gumbel_from_bits(bits):
    fbits = lax.shift_right_logical(bits, jnp.int32(9)) | jnp.int32(0x3F800000)
    f = lax.bitcast_convert_type(fbits, jnp.float32) - np.float32(1.0)
    span = np.float32(np.float32(1.0) - _TINY)
    u = jnp.maximum(_TINY, f * span + _TINY)
    return -jnp.log(-jnp.log(u))


def _stats_kernel(z_ref, mu_ref, inv_ref):
    blk = z_ref[...]
    b3 = blk.reshape(128, 8, 2048)
    s = jnp.sum(b3, axis=2)
    q = jnp.sum(b3 * b3, axis=2)
    s_t = s.T
    q_t = q.T
    inv_n = jnp.float32(1.0 / 2048.0)
    mu = s_t * inv_n
    var = q_t * inv_n - mu * mu
    den = jnp.sqrt(jnp.maximum(var, 0.0)) + jnp.float32(0.001)
    mu_ref[0] = mu
    inv_ref[0] = jnp.float32(1.0) / den


def _sample_kernel(z_ref, mu_ref, inv_ref, out_ref):
    b0 = pl.program_id(0) * 128
    a_i = lax.broadcasted_iota(jnp.int32, (128, 8, 128), 0)
    s_i = lax.broadcasted_iota(jnp.int32, (128, 8, 128), 1)
    l_i = lax.broadcasted_iota(jnp.int32, (128, 8, 128), 2)
    ibase = (b0 + a_i) * jnp.int32(16384) + l_i * jnp.int32(8) + s_i
    jbase = l_i * jnp.int32(8) + s_i
    cols = []
    for hq in range(8):
        m = jnp.full((128, 8, 128), -jnp.inf, jnp.float32)
        jw = jnp.zeros((128, 8, 128), jnp.int32)
        for cc in range(2):
            c = hq * 2 + cc
            v3 = z_ref[:, c * 128:(c + 1) * 128].reshape(128, 8, 128)
            mu_t = mu_ref[c].reshape(1, 8, 128)
            inv_t = inv_ref[c].reshape(1, 8, 128)
            std = (v3 - mu_t) * inv_t
            lin = ibase + jnp.int32(2048 * hq + 1024 * cc)
            g = _gumbel_from_bits(_threefry_bits(lin))
            val = std + g
            upd = val > m
            m = jnp.where(upd, val, m)
            jw = jnp.where(upd, jbase + jnp.int32(1024 * cc), jw)
        mx = jnp.max(jnp.max(m, axis=2, keepdims=True), axis=1, keepdims=True)
        cand = jnp.where(m == mx, jw, jnp.int32(0x7FFFFFFF))
        cols.append(jnp.min(jnp.min(cand, axis=2), axis=1, keepdims=True))
    out_ref[...] = jnp.concatenate(cols, axis=1)


@jax.jit
def kernel(x):
    B, H, W = x.shape
    z = x.transpose(0, 2, 1).reshape(B * W, H)
    mu_t, inv_t = pl.pallas_call(
        _stats_kernel,
        grid=(16,),
        in_specs=[pl.BlockSpec((1024, H), lambda k: (k, 0))],
        out_specs=[
            pl.BlockSpec((1, 8, 128), lambda k: (k, 0, 0)),
            pl.BlockSpec((1, 8, 128), lambda k: (k, 0, 0)),
        ],
        out_shape=[
            jax.ShapeDtypeStruct((16, 8, 128), jnp.float32),
            jax.ShapeDtypeStruct((16, 8, 128), jnp.float32),
        ],
    )(z)
    out = pl.pallas_call(
        _sample_kernel,
        grid=(16,),
        in_specs=[
            pl.BlockSpec((1024, H), lambda k: (k, 0)),
            pl.BlockSpec((16, 8, 128), lambda k: (0, 0, 0)),
            pl.BlockSpec((16, 8, 128), lambda k: (0, 0, 0)),
        ],
        out_specs=pl.BlockSpec((128, 8), lambda k: (k, 0)),
        out_shape=jax.ShapeDtypeStruct((B, W), jnp.int32),
    )(z, mu_t, inv_t)
    return out.reshape(1, H, W)


# speedup vs baseline: 3.2182x; 1.0119x over previous
import jax
import jax.numpy as jnp
import numpy as np
from jax import lax
from jax.experimental import pallas as pl
from jax.experimental.pallas import tpu as pltpu

_TINY = np.float32(1.1754943508222875e-38)
_K0 = 0
_K1 = 42
_K2 = _K0 ^ _K1 ^ 0x1BD11BDA


def _rotl(x, d):
    return lax.shift_left(x, jnp.int32(d)) | lax.shift_right_logical(
        x, jnp.int32(32 - d))


def _threefry_bits(i_plus_k1):
    ks = [jnp.int32(_K0), jnp.int32(_K1), jnp.int32(_K2)]
    rot0 = (13, 15, 26, 6)
    rot1 = (17, 29, 16, 24)
    x0 = jnp.zeros_like(i_plus_k1) + ks[0]
    x1 = i_plus_k1

    def rounds(x0, x1, rots):
        for r in rots:
            x0 = x0 + x1
            x1 = x0 ^ _rotl(x1, r)
        return x0, x1

    x0, x1 = rounds(x0, x1, rot0)
    x0 = x0 + ks[1]
    x1 = x1 + (ks[2] + jnp.int32(1))
    x0, x1 = rounds(x0, x1, rot1)
    x0 = x0 + ks[2]
    x1 = x1 + (ks[0] + jnp.int32(2))
    x0, x1 = rounds(x0, x1, rot0)
    x0 = x0 + ks[0]
    x1 = x1 + (ks[1] + jnp.int32(3))
    x0, x1 = rounds(x0, x1, rot1)
    x0 = x0 + ks[1]
    x1 = x1 + (ks[2] + jnp.int32(4))
    x0, x1 = rounds(x0, x1, rot0)
    x0 = x0 + ks[2]
    x1 = x1 + (ks[0] + jnp.int32(5))
    return x0 ^ x1


def _gumbel_from_bits(bits):
    fbits = lax.shift_right_logical(bits, jnp.int32(9)) | jnp.int32(0x3F800000)
    f = lax.bitcast_convert_type(fbits, jnp.float32) - np.float32(1.0)
    u = f + _TINY
    return -jnp.log(-jnp.log(u))


def _stats_kernel(z_ref, mu_ref, inv_ref):
    blk = z_ref[...]
    b3 = blk.reshape(128, 8, 2048)
    s = jnp.sum(b3, axis=2)
    q = jnp.sum(b3 * b3, axis=2)
    s_t = s.T
    q_t = q.T
    inv_n = jnp.float32(1.0 / 2048.0)
    mu = s_t * inv_n
    var = q_t * inv_n - mu * mu
    den = jnp.sqrt(jnp.maximum(var, 0.0)) + jnp.float32(0.001)
    mu_ref[0] = mu
    inv_ref[0] = jnp.float32(1.0) / den


def _sample_kernel(z_ref, mu_ref, inv_ref, out_ref):
    b0 = pl.program_id(0) * 128
    a_i = lax.broadcasted_iota(jnp.int32, (128, 8, 128), 0)
    s_i = lax.broadcasted_iota(jnp.int32, (128, 8, 128), 1)
    l_i = lax.broadcasted_iota(jnp.int32, (128, 8, 128), 2)
    ibase = (b0 + a_i) * jnp.int32(16384) + l_i * jnp.int32(8) + s_i + jnp.int32(_K1)
    jbase = l_i * jnp.int32(8) + s_i
    cols = []
    for hq in range(8):
        m = jnp.full((128, 8, 128), -jnp.inf, jnp.float32)
        jw = jnp.zeros((128, 8, 128), jnp.int32)
        for cc in range(2):
            c = hq * 2 + cc
            v3 = z_ref[:, c * 128:(c + 1) * 128].reshape(128, 8, 128)
            mu_t = mu_ref[c].reshape(1, 8, 128)
            inv_t = inv_ref[c].reshape(1, 8, 128)
            std = (v3 - mu_t) * inv_t
            lin = ibase + jnp.int32(2048 * hq + 1024 * cc)
            g = _gumbel_from_bits(_threefry_bits(lin))
            val = std + g
            upd = val > m
            m = jnp.where(upd, val, m)
            jw = jnp.where(upd, jbase + jnp.int32(1024 * cc), jw)
        mx = jnp.max(jnp.max(m, axis=2, keepdims=True), axis=1, keepdims=True)
        cand = jnp.where(m == mx, jw, jnp.int32(0x7FFFFFFF))
        cols.append(jnp.min(jnp.min(cand, axis=2), axis=1, keepdims=True))
    out_ref[...] = jnp.concatenate(cols, axis=1)


@jax.jit
def kernel(x):
    B, H, W = x.shape
    z = x.transpose(0, 2, 1).reshape(B * W, H)
    mu_t, inv_t = pl.pallas_call(
        _stats_kernel,
        grid=(16,),
        in_specs=[pl.BlockSpec((1024, H), lambda k: (k, 0))],
        out_specs=[
            pl.BlockSpec((1, 8, 128), lambda k: (k, 0, 0)),
            pl.BlockSpec((1, 8, 128), lambda k: (k, 0, 0)),
        ],
        out_shape=[
            jax.ShapeDtypeStruct((16, 8, 128), jnp.float32),
            jax.ShapeDtypeStruct((16, 8, 128), jnp.float32),
        ],
        compiler_params=pltpu.CompilerParams(
            dimension_semantics=("parallel",)),
    )(z)
    out = pl.pallas_call(
        _sample_kernel,
        grid=(16,),
        in_specs=[
            pl.BlockSpec((1024, H), lambda k: (k, 0)),
            pl.BlockSpec((16, 8, 128), lambda k: (0, 0, 0)),
            pl.BlockSpec((16, 8, 128), lambda k: (0, 0, 0)),
        ],
        out_specs=pl.BlockSpec((128, 8), lambda k: (k, 0)),
        out_shape=jax.ShapeDtypeStruct((B, W), jnp.int32),
        compiler_params=pltpu.CompilerParams(
            dimension_semantics=("parallel",)),
    )(z, mu_t, inv_t)
    return out.reshape(1, H, W)


# speedup vs baseline: 3.9075x; 1.2142x over previous
import jax
import jax.numpy as jnp
import numpy as np
from jax import lax
from jax.experimental import pallas as pl
from jax.experimental.pallas import tpu as pltpu

_TINY = np.float32(1.1754943508222875e-38)
_SB = 32
_K0 = 0
_K1 = 42
_K2 = _K0 ^ _K1 ^ 0x1BD11BDA


def _rotl(x, d):
    return lax.shift_left(x, jnp.int32(d)) | lax.shift_right_logical(
        x, jnp.int32(32 - d))


def _threefry_bits(i_plus_k1):
    ks = [jnp.int32(_K0), jnp.int32(_K1), jnp.int32(_K2)]
    rot0 = (13, 15, 26, 6)
    rot1 = (17, 29, 16, 24)
    x0 = jnp.zeros_like(i_plus_k1) + ks[0]
    x1 = i_plus_k1

    def rounds(x0, x1, rots):
        for r in rots:
            x0 = x0 + x1
            x1 = x0 ^ _rotl(x1, r)
        return x0, x1

    x0, x1 = rounds(x0, x1, rot0)
    x0 = x0 + ks[1]
    x1 = x1 + (ks[2] + jnp.int32(1))
    x0, x1 = rounds(x0, x1, rot1)
    x0 = x0 + ks[2]
    x1 = x1 + (ks[0] + jnp.int32(2))
    x0, x1 = rounds(x0, x1, rot0)
    x0 = x0 + ks[0]
    x1 = x1 + (ks[1] + jnp.int32(3))
    x0, x1 = rounds(x0, x1, rot1)
    x0 = x0 + ks[1]
    x1 = x1 + (ks[2] + jnp.int32(4))
    x0, x1 = rounds(x0, x1, rot0)
    x0 = x0 + ks[2]
    x1 = x1 + (ks[0] + jnp.int32(5))
    return x0 ^ x1


def _gumbel_from_bits(bits):
    fbits = lax.shift_right_logical(bits, jnp.int32(9)) | jnp.int32(0x3F800000)
    f = lax.bitcast_convert_type(fbits, jnp.float32) - np.float32(1.0)
    u = f + _TINY
    return -jnp.log(-jnp.log(u))


def _stats_kernel(z_ref, mu_ref, inv_ref):
    blk = z_ref[...]
    b3 = blk.reshape(128, 8, 2048)
    s = jnp.sum(b3, axis=2)
    q = jnp.sum(b3 * b3, axis=2)
    s_t = s.T
    q_t = q.T
    inv_n = jnp.float32(1.0 / 2048.0)
    mu = s_t * inv_n
    var = q_t * inv_n - mu * mu
    den = jnp.sqrt(jnp.maximum(var, 0.0)) + jnp.float32(0.001)
    mu_ref[0] = mu
    inv_ref[0] = jnp.float32(1.0) / den


def _sample_kernel(z_ref, mu_ref, inv_ref, out_ref):
    b0 = pl.program_id(0) * _SB
    a_i = lax.broadcasted_iota(jnp.int32, (_SB, 8, 128), 0)
    s_i = lax.broadcasted_iota(jnp.int32, (_SB, 8, 128), 1)
    l_i = lax.broadcasted_iota(jnp.int32, (_SB, 8, 128), 2)
    ibase = (b0 + a_i) * jnp.int32(16384) + l_i * jnp.int32(8) + s_i + jnp.int32(_K1)
    jbase = l_i * jnp.int32(8) + s_i
    cols = []
    for hq in range(8):
        m = jnp.full((_SB, 8, 128), -jnp.inf, jnp.float32)
        jw = jnp.zeros((_SB, 8, 128), jnp.int32)
        for cc in range(2):
            c = hq * 2 + cc
            v3 = z_ref[:, c * 128:(c + 1) * 128].reshape(_SB, 8, 128)
            mu_t = mu_ref[c].reshape(1, 8, 128)
            inv_t = inv_ref[c].reshape(1, 8, 128)
            std = (v3 - mu_t) * inv_t
            lin = ibase + jnp.int32(2048 * hq + 1024 * cc)
            g = _gumbel_from_bits(_threefry_bits(lin))
            val = std + g
            upd = val > m
            m = jnp.where(upd, val, m)
            jw = jnp.where(upd, jbase + jnp.int32(1024 * cc), jw)
        mx = jnp.max(jnp.max(m, axis=2, keepdims=True), axis=1, keepdims=True)
        cand = jnp.where(m == mx, jw, jnp.int32(0x7FFFFFFF))
        cols.append(jnp.min(jnp.min(cand, axis=2), axis=1, keepdims=True))
    out_ref[...] = jnp.concatenate(cols, axis=1)


@jax.jit
def kernel(x):
    B, H, W = x.shape
    z = x.transpose(0, 2, 1).reshape(B * W, H)
    mu_t, inv_t = pl.pallas_call(
        _stats_kernel,
        grid=(16,),
        in_specs=[pl.BlockSpec((1024, H), lambda k: (k, 0))],
        out_specs=[
            pl.BlockSpec((1, 8, 128), lambda k: (k, 0, 0)),
            pl.BlockSpec((1, 8, 128), lambda k: (k, 0, 0)),
        ],
        out_shape=[
            jax.ShapeDtypeStruct((16, 8, 128), jnp.float32),
            jax.ShapeDtypeStruct((16, 8, 128), jnp.float32),
        ],
        compiler_params=pltpu.CompilerParams(
            dimension_semantics=("parallel",)),
    )(z)
    out = pl.pallas_call(
        _sample_kernel,
        grid=(B // _SB,),
        in_specs=[
            pl.BlockSpec((8 * _SB, H), lambda k: (k, 0)),
            pl.BlockSpec((16, 8, 128), lambda k: (0, 0, 0)),
            pl.BlockSpec((16, 8, 128), lambda k: (0, 0, 0)),
        ],
        out_specs=pl.BlockSpec((_SB, 8), lambda k: (k, 0)),
        out_shape=jax.ShapeDtypeStruct((B, W), jnp.int32),
        compiler_params=pltpu.CompilerParams(
            dimension_semantics=("parallel",)),
    )(z, mu_t, inv_t)
    return out.reshape(1, H, W)


# speedup vs baseline: 3.9667x; 1.0151x over previous
import jax
import jax.numpy as jnp
import numpy as np
from jax import lax
from jax.experimental import pallas as pl
from jax.experimental.pallas import tpu as pltpu

_TINY = np.float32(1.1754943508222875e-38)
_SB = 16
_K0 = 0
_K1 = 42
_K2 = _K0 ^ _K1 ^ 0x1BD11BDA


def _rotl(x, d):
    return lax.shift_left(x, jnp.int32(d)) | lax.shift_right_logical(
        x, jnp.int32(32 - d))


def _threefry_bits(i_plus_k1):
    ks = [jnp.int32(_K0), jnp.int32(_K1), jnp.int32(_K2)]
    rot0 = (13, 15, 26, 6)
    rot1 = (17, 29, 16, 24)
    x0 = jnp.zeros_like(i_plus_k1) + ks[0]
    x1 = i_plus_k1

    def rounds(x0, x1, rots):
        for r in rots:
            x0 = x0 + x1
            x1 = x0 ^ _rotl(x1, r)
        return x0, x1

    x0, x1 = rounds(x0, x1, rot0)
    x0 = x0 + ks[1]
    x1 = x1 + (ks[2] + jnp.int32(1))
    x0, x1 = rounds(x0, x1, rot1)
    x0 = x0 + ks[2]
    x1 = x1 + (ks[0] + jnp.int32(2))
    x0, x1 = rounds(x0, x1, rot0)
    x0 = x0 + ks[0]
    x1 = x1 + (ks[1] + jnp.int32(3))
    x0, x1 = rounds(x0, x1, rot1)
    x0 = x0 + ks[1]
    x1 = x1 + (ks[2] + jnp.int32(4))
    x0, x1 = rounds(x0, x1, rot0)
    x0 = x0 + ks[2]
    x1 = x1 + (ks[0] + jnp.int32(5))
    return x0 ^ x1


def _gumbel_from_bits(bits):
    fbits = lax.shift_right_logical(bits, jnp.int32(9)) | jnp.int32(0x3F800000)
    f = lax.bitcast_convert_type(fbits, jnp.float32) - np.float32(1.0)
    u = f + _TINY
    return -jnp.log(-jnp.log(u))


def _stats_kernel(z_ref, mu_ref, inv_ref):
    blk = z_ref[...]
    b3 = blk.reshape(128, 8, 2048)
    s = jnp.sum(b3, axis=2)
    q = jnp.sum(b3 * b3, axis=2)
    s_t = s.T
    q_t = q.T
    inv_n = jnp.float32(1.0 / 2048.0)
    mu = s_t * inv_n
    var = q_t * inv_n - mu * mu
    den = jnp.sqrt(jnp.maximum(var, 0.0)) + jnp.float32(0.001)
    mu_ref[0] = mu
    inv_ref[0] = jnp.float32(1.0) / den


def _sample_kernel(z_ref, mu_ref, inv_ref, out_ref):
    b0 = pl.program_id(0) * _SB
    a_i = lax.broadcasted_iota(jnp.int32, (_SB, 8, 128), 0)
    s_i = lax.broadcasted_iota(jnp.int32, (_SB, 8, 128), 1)
    l_i = lax.broadcasted_iota(jnp.int32, (_SB, 8, 128), 2)
    ibase = (b0 + a_i) * jnp.int32(16384) + l_i * jnp.int32(8) + s_i + jnp.int32(_K1)
    jbase = l_i * jnp.int32(8) + s_i
    cols = []
    for hq in range(8):
        m = jnp.full((_SB, 8, 128), -jnp.inf, jnp.float32)
        jw = jnp.zeros((_SB, 8, 128), jnp.int32)
        for cc in range(2):
            c = hq * 2 + cc
            v3 = z_ref[:, c * 128:(c + 1) * 128].reshape(_SB, 8, 128)
            mu_t = mu_ref[c].reshape(1, 8, 128)
            inv_t = inv_ref[c].reshape(1, 8, 128)
            std = (v3 - mu_t) * inv_t
            lin = ibase + jnp.int32(2048 * hq + 1024 * cc)
            g = _gumbel_from_bits(_threefry_bits(lin))
            val = std + g
            upd = val > m
            m = jnp.where(upd, val, m)
            jw = jnp.where(upd, jbase + jnp.int32(1024 * cc), jw)
        mx = jnp.max(jnp.max(m, axis=2, keepdims=True), axis=1, keepdims=True)
        cand = jnp.where(m == mx, jw, jnp.int32(0x7FFFFFFF))
        cols.append(jnp.min(jnp.min(cand, axis=2), axis=1, keepdims=True))
    out_ref[...] = jnp.concatenate(cols, axis=1)


@jax.jit
def kernel(x):
    B, H, W = x.shape
    z = x.transpose(0, 2, 1).reshape(B * W, H)
    mu_t, inv_t = pl.pallas_call(
        _stats_kernel,
        grid=(16,),
        in_specs=[pl.BlockSpec((1024, H), lambda k: (k, 0))],
        out_specs=[
            pl.BlockSpec((1, 8, 128), lambda k: (k, 0, 0)),
            pl.BlockSpec((1, 8, 128), lambda k: (k, 0, 0)),
        ],
        out_shape=[
            jax.ShapeDtypeStruct((16, 8, 128), jnp.float32),
            jax.ShapeDtypeStruct((16, 8, 128), jnp.float32),
        ],
        compiler_params=pltpu.CompilerParams(
            dimension_semantics=("parallel",)),
    )(z)
    out = pl.pallas_call(
        _sample_kernel,
        grid=(B // _SB,),
        in_specs=[
            pl.BlockSpec((8 * _SB, H), lambda k: (k, 0)),
            pl.BlockSpec((16, 8, 128), lambda k: (0, 0, 0)),
            pl.BlockSpec((16, 8, 128), lambda k: (0, 0, 0)),
        ],
        out_specs=pl.BlockSpec((_SB, 8), lambda k: (k, 0)),
        out_shape=jax.ShapeDtypeStruct((B, W), jnp.int32),
        compiler_params=pltpu.CompilerParams(
            dimension_semantics=("parallel",)),
    )(z, mu_t, inv_t)
    return out.reshape(1, H, W)
